# Initial kernel scaffold; baseline (speedup 1.0000x reference)
import functools

import jax
import jax.numpy as jnp
from jax import lax
from jax.experimental import pallas as pl
from jax.experimental.pallas import tpu as pltpu
from jax.experimental.pallas import tpu_sc as plsc

N = 10000
E = 320000
P = 100000
F_IN = 128
H = 64

NC = 2
NS = 16
NW = NC * NS

N_PAD = 10240
E_BLOCKS = 80
E_PAD = NW * E_BLOCKS * 128
P_BLOCKS = 25
P_PAD = NW * P_BLOCKS * 128

ROWS_PER_SUB = N_PAD // NS


def _mesh():
    return plsc.VectorSubcoreMesh(core_axis_name="c", subcore_axis_name="s")


@functools.partial(
    pl.kernel,
    out_type=jax.ShapeDtypeStruct((NC * N_PAD, 16), jnp.float32),
    mesh=_mesh(),
    scratch_types=[
        pltpu.VMEM((E_BLOCKS, 128), jnp.int32),
        pltpu.VMEM((128, 16), jnp.float32),
        pltpu.VMEM_SHARED((N_PAD, 16), jnp.float32),
    ],
)
def _deg_kernel(dst_hbm, ones_hbm, zeros_hbm, out_hbm, didx, ones_v, acc):
    c = lax.axis_index("c")
    s = lax.axis_index("s")
    wid = s * NC + c
    r0 = s * ROWS_PER_SUB
    pltpu.sync_copy(zeros_hbm.at[pl.ds(r0, ROWS_PER_SUB)],
                    acc.at[pl.ds(r0, ROWS_PER_SUB)])
    pltpu.sync_copy(ones_hbm, ones_v)
    pltpu.sync_copy(dst_hbm.at[wid], didx)
    plsc.subcore_barrier()

    def body(j, carry):
        pltpu.sync_copy(ones_v, acc.at[didx.at[j]], add=True)
        return carry

    lax.fori_loop(0, E_BLOCKS, body, 0)
    plsc.subcore_barrier()
    pltpu.sync_copy(acc.at[pl.ds(r0, ROWS_PER_SUB)],
                    out_hbm.at[pl.ds(c * N_PAD + r0, ROWS_PER_SUB)])


@functools.partial(
    pl.kernel,
    out_type=jax.ShapeDtypeStruct((NC * N_PAD, H), jnp.float32),
    mesh=_mesh(),
    scratch_types=[
        pltpu.VMEM((E_BLOCKS, 128), jnp.int32),
        pltpu.VMEM((E_BLOCKS, 128), jnp.int32),
        pltpu.VMEM((128, H), jnp.float32),
        pltpu.VMEM_SHARED((N_PAD, H), jnp.float32),
        pltpu.SemaphoreType.DMA,
    ],
)
def _edge_scatter_kernel(q_hbm, src_hbm, dst_hbm, zeros_hbm, out_hbm,
                         sidx, didx, rows, acc, sem):
    c = lax.axis_index("c")
    s = lax.axis_index("s")
    wid = s * NC + c
    r0 = s * ROWS_PER_SUB
    pltpu.sync_copy(zeros_hbm.at[pl.ds(r0, ROWS_PER_SUB)],
                    acc.at[pl.ds(r0, ROWS_PER_SUB)])
    pltpu.sync_copy(src_hbm.at[wid], sidx)
    pltpu.sync_copy(dst_hbm.at[wid], didx)
    plsc.subcore_barrier()

    def body(j, carry):
        pltpu.async_copy(q_hbm.at[sidx.at[j]], rows, sem).wait()
        pltpu.sync_copy(rows, acc.at[didx.at[j]], add=True)
        return carry

    lax.fori_loop(0, E_BLOCKS, body, 0)
    plsc.subcore_barrier()
    pltpu.sync_copy(acc.at[pl.ds(r0, ROWS_PER_SUB)],
                    out_hbm.at[pl.ds(c * N_PAD + r0, ROWS_PER_SUB)])


@functools.partial(
    pl.kernel,
    out_type=(jax.ShapeDtypeStruct((P_PAD, H), jnp.float32),
              jax.ShapeDtypeStruct((P_PAD, H), jnp.float32)),
    mesh=_mesh(),
    scratch_types=[
        pltpu.VMEM((P_BLOCKS, 128), jnp.int32),
        pltpu.VMEM((P_BLOCKS, 128), jnp.int32),
        pltpu.VMEM((128, H), jnp.float32),
        pltpu.VMEM((128, H), jnp.float32),
        pltpu.SemaphoreType.DMA,
        pltpu.SemaphoreType.DMA,
    ],
)
def _flow_gather_kernel(a_hbm, b_hbm, fsrc_hbm, ftgt_hbm, gs_hbm, gt_hbm,
                        sidx, tidx, rows_a, rows_b, sem_a, sem_b):
    c = lax.axis_index("c")
    s = lax.axis_index("s")
    wid = s * NC + c
    base = wid * P_BLOCKS * 128
    pltpu.sync_copy(fsrc_hbm.at[wid], sidx)
    pltpu.sync_copy(ftgt_hbm.at[wid], tidx)

    def body(j, carry):
        da = pltpu.async_copy(a_hbm.at[sidx.at[j]], rows_a, sem_a)
        db = pltpu.async_copy(b_hbm.at[tidx.at[j]], rows_b, sem_b)
        da.wait()
        pltpu.sync_copy(rows_a, gs_hbm.at[pl.ds(base + j * 128, 128)])
        db.wait()
        pltpu.sync_copy(rows_b, gt_hbm.at[pl.ds(base + j * 128, 128)])
        return carry

    lax.fori_loop(0, P_BLOCKS, body, 0)


TC_BLK = 512
TC_GRID = N_PAD // TC_BLK


def _enc_body(x_ref, degp_ref, we_ref, be_ref, w1_ref, q1_ref, dinv_ref):
    cnt = degp_ref[0][:, 0:1] + degp_ref[1][:, 0:1]
    dinv = lax.rsqrt(cnt + 1.0)
    h0 = jax.nn.relu(
        jnp.dot(x_ref[...], we_ref[...], preferred_element_type=jnp.float32)
        + be_ref[...])
    q1_ref[...] = jnp.dot(h0, w1_ref[...],
                          preferred_element_type=jnp.float32) * dinv
    dinv_ref[...] = jnp.broadcast_to(dinv, (TC_BLK, 8))


def _tc_encoder(x_pad, degp, W_enc, b_enc, W1):
    return pl.pallas_call(
        _enc_body,
        grid=(TC_GRID,),
        in_specs=[
            pl.BlockSpec((TC_BLK, F_IN), lambda i: (i, 0)),
            pl.BlockSpec((NC, TC_BLK, 16), lambda i: (0, i, 0)),
            pl.BlockSpec((F_IN, H), lambda i: (0, 0)),
            pl.BlockSpec((1, H), lambda i: (0, 0)),
            pl.BlockSpec((H, H), lambda i: (0, 0)),
        ],
        out_specs=[
            pl.BlockSpec((TC_BLK, H), lambda i: (i, 0)),
            pl.BlockSpec((TC_BLK, 8), lambda i: (i, 0)),
        ],
        out_shape=[
            jax.ShapeDtypeStruct((N_PAD, H), jnp.float32),
            jax.ShapeDtypeStruct((N_PAD, 8), jnp.float32),
        ],
    )(x_pad, degp, W_enc, b_enc.reshape(1, H), W1)


def _layer_body(sp_ref, q_ref, dinv_ref, b_ref, wn_ref, qn_ref):
    dinv = dinv_ref[:, 0:1]
    h = jax.nn.relu(
        (sp_ref[0] + sp_ref[1] + q_ref[...]) * dinv + b_ref[...])
    qn_ref[...] = jnp.dot(h, wn_ref[...],
                          preferred_element_type=jnp.float32) * dinv


def _tc_layer(sp, q, dinv, b, W_next):
    return pl.pallas_call(
        _layer_body,
        grid=(TC_GRID,),
        in_specs=[
            pl.BlockSpec((NC, TC_BLK, H), lambda i: (0, i, 0)),
            pl.BlockSpec((TC_BLK, H), lambda i: (i, 0)),
            pl.BlockSpec((TC_BLK, 8), lambda i: (i, 0)),
            pl.BlockSpec((1, H), lambda i: (0, 0)),
            pl.BlockSpec((H, H), lambda i: (0, 0)),
        ],
        out_specs=pl.BlockSpec((TC_BLK, H), lambda i: (i, 0)),
        out_shape=jax.ShapeDtypeStruct((N_PAD, H), jnp.float32),
    )(sp, q, dinv, b.reshape(1, H), W_next)


def _proj_body(sp_ref, q_ref, dinv_ref, b_ref, wt_ref, wb_ref, a_ref, b2_ref):
    dinv = dinv_ref[:, 0:1]
    h = jax.nn.relu(
        (sp_ref[0] + sp_ref[1] + q_ref[...]) * dinv + b_ref[...])
    a_ref[...] = jnp.dot(h, wt_ref[...], preferred_element_type=jnp.float32)
    b2_ref[...] = jnp.dot(h, wb_ref[...], preferred_element_type=jnp.float32)


def _tc_proj(sp, q, dinv, b, Wf1):
    return pl.pallas_call(
        _proj_body,
        grid=(TC_GRID,),
        in_specs=[
            pl.BlockSpec((NC, TC_BLK, H), lambda i: (0, i, 0)),
            pl.BlockSpec((TC_BLK, H), lambda i: (i, 0)),
            pl.BlockSpec((TC_BLK, 8), lambda i: (i, 0)),
            pl.BlockSpec((1, H), lambda i: (0, 0)),
            pl.BlockSpec((H, H), lambda i: (0, 0)),
            pl.BlockSpec((H, H), lambda i: (0, 0)),
        ],
        out_specs=[
            pl.BlockSpec((TC_BLK, H), lambda i: (i, 0)),
            pl.BlockSpec((TC_BLK, H), lambda i: (i, 0)),
        ],
        out_shape=[
            jax.ShapeDtypeStruct((N_PAD, H), jnp.float32),
            jax.ShapeDtypeStruct((N_PAD, H), jnp.float32),
        ],
    )(sp, q, dinv, b.reshape(1, H), Wf1[:H], Wf1[H:])


MLP_BLK = 1000
MLP_GRID = P // MLP_BLK


def _mlp_body(gs_ref, gt_ref, b1_ref, w2_ref, b2_ref, w3_ref, b3_ref, out_ref):
    z = jax.nn.relu(gs_ref[...] + gt_ref[...] + b1_ref[...])
    z2 = jax.nn.relu(
        jnp.dot(z, w2_ref[...], preferred_element_type=jnp.float32)
        + b2_ref[...])
    out_ref[...] = jnp.dot(z2, w3_ref[...],
                           preferred_element_type=jnp.float32) + b3_ref[...]


def _tc_mlp(gs, gt, bf1, Wf2, bf2, Wf3, bf3):
    return pl.pallas_call(
        _mlp_body,
        grid=(MLP_GRID,),
        in_specs=[
            pl.BlockSpec((MLP_BLK, H), lambda i: (i, 0)),
            pl.BlockSpec((MLP_BLK, H), lambda i: (i, 0)),
            pl.BlockSpec((1, H), lambda i: (0, 0)),
            pl.BlockSpec((H, H // 2), lambda i: (0, 0)),
            pl.BlockSpec((1, H // 2), lambda i: (0, 0)),
            pl.BlockSpec((H // 2, 1), lambda i: (0, 0)),
            pl.BlockSpec((1, 1), lambda i: (0, 0)),
        ],
        out_specs=pl.BlockSpec((MLP_BLK, 1), lambda i: (i, 0)),
        out_shape=jax.ShapeDtypeStruct((P, 1), jnp.float32),
    )(gs, gt, bf1.reshape(1, H), Wf2, bf2.reshape(1, H // 2), Wf3,
      bf3.reshape(1, 1))


def kernel(x, edge_index, flow_edges, W_enc, b_enc, W1, b1, W2, b2, W3, b3,
           Wf1, bf1, Wf2, bf2, Wf3, bf3):
    x_pad = jnp.concatenate(
        [x, jnp.zeros((N_PAD - N, F_IN), jnp.float32)], axis=0)

    def pad_idx(idx, length, blocks, fill):
        idx = idx.astype(jnp.int32)
        idx = jnp.concatenate(
            [idx, jnp.full((NW * blocks * 128 - length,), fill, jnp.int32)])
        return idx.reshape(NW, blocks, 128)

    src3 = pad_idx(edge_index[0], E, E_BLOCKS, N)
    dst3 = pad_idx(edge_index[1], E, E_BLOCKS, N)
    fsrc3 = pad_idx(flow_edges[0], P, P_BLOCKS, 0)
    ftgt3 = pad_idx(flow_edges[1], P, P_BLOCKS, 0)

    zeros16 = jnp.zeros((N_PAD, 16), jnp.float32)
    zeros64 = jnp.zeros((N_PAD, H), jnp.float32)
    ones16 = jnp.ones((128, 16), jnp.float32)

    degp = _deg_kernel(dst3, ones16, zeros16).reshape(NC, N_PAD, 16)

    q1, dinv = _tc_encoder(x_pad, degp, W_enc, b_enc, W1)

    s1 = _edge_scatter_kernel(q1, src3, dst3, zeros64).reshape(NC, N_PAD, H)
    q2 = _tc_layer(s1, q1, dinv, b1, W2)
    s2 = _edge_scatter_kernel(q2, src3, dst3, zeros64).reshape(NC, N_PAD, H)
    q3 = _tc_layer(s2, q2, dinv, b2, W3)
    s3 = _edge_scatter_kernel(q3, src3, dst3, zeros64).reshape(NC, N_PAD, H)

    A, B = _tc_proj(s3, q3, dinv, b3, Wf1)
    gs, gt = _flow_gather_kernel(A, B, fsrc3, ftgt3)
    flows = _tc_mlp(gs[:P], gt[:P], bf1, Wf2, bf2, Wf3, bf3)
    return flows


# speedup vs baseline: 5.4786x; 5.4786x over previous
import functools

import jax
import jax.numpy as jnp
from jax import lax
from jax.experimental import pallas as pl
from jax.experimental.pallas import tpu as pltpu
from jax.experimental.pallas import tpu_sc as plsc

N = 10000
E = 320000
P = 100000
F_IN = 128
H = 64

NC = 2
NS = 16
NW = NC * NS

N_PAD = 10240
E_BLOCKS = 80
E_PAD = NW * E_BLOCKS * 128
P_BLOCKS = 25
P_PAD = NW * P_BLOCKS * 128

ROWS_PER_SUB = N_PAD // NS


def _mesh():
    return plsc.VectorSubcoreMesh(core_axis_name="c", subcore_axis_name="s")


def _deg_body(dst_hbm, ones_hbm, zeros_hbm, out_hbm, didx, ones_v, acc):
    c = lax.axis_index("c")
    s = lax.axis_index("s")
    wid = s * NC + c
    r0 = s * ROWS_PER_SUB
    pltpu.sync_copy(zeros_hbm.at[pl.ds(r0, ROWS_PER_SUB)],
                    acc.at[pl.ds(r0, ROWS_PER_SUB)])
    pltpu.sync_copy(ones_hbm, ones_v)
    pltpu.sync_copy(dst_hbm.at[wid], didx)
    plsc.subcore_barrier()

    def body(j, carry):
        pltpu.sync_copy(ones_v, acc.at[didx.at[j]], add=True)
        return carry

    lax.fori_loop(0, E_BLOCKS, body, 0)
    plsc.subcore_barrier()
    pltpu.sync_copy(acc.at[pl.ds(r0, ROWS_PER_SUB)],
                    out_hbm.at[pl.ds(c * N_PAD + r0, ROWS_PER_SUB)])


_deg_kernel = pl.kernel(
    _deg_body,
    out_type=jax.ShapeDtypeStruct((NC * N_PAD, 2 * H), jnp.float32),
    mesh=_mesh(),
    scratch_types=[
        pltpu.VMEM((E_BLOCKS, 128), jnp.int32),
        pltpu.VMEM((128, 2 * H), jnp.float32),
        pltpu.VMEM_SHARED((N_PAD, 2 * H), jnp.float32),
    ],
)


def _edge_scatter_body(q_hbm, src_hbm, dst_hbm, zeros_hbm, out_hbm,
                         sidx, didx, rows, acc, sem):
    c = lax.axis_index("c")
    s = lax.axis_index("s")
    wid = s * NC + c
    r0 = s * ROWS_PER_SUB
    pltpu.sync_copy(zeros_hbm.at[pl.ds(r0, ROWS_PER_SUB)],
                    acc.at[pl.ds(r0, ROWS_PER_SUB)])
    pltpu.sync_copy(src_hbm.at[wid], sidx)
    pltpu.sync_copy(dst_hbm.at[wid], didx)
    plsc.subcore_barrier()

    def body(j, carry):
        pltpu.async_copy(q_hbm.at[sidx.at[j]], rows, sem).wait()
        pltpu.sync_copy(rows, acc.at[didx.at[j]], add=True)
        return carry

    lax.fori_loop(0, E_BLOCKS, body, 0)
    plsc.subcore_barrier()
    pltpu.sync_copy(acc.at[pl.ds(r0, ROWS_PER_SUB)],
                    out_hbm.at[pl.ds(c * N_PAD + r0, ROWS_PER_SUB)])


_edge_scatter_kernel = pl.kernel(
    _edge_scatter_body,
    out_type=jax.ShapeDtypeStruct((NC * N_PAD, 2 * H), jnp.float32),
    mesh=_mesh(),
    scratch_types=[
        pltpu.VMEM((E_BLOCKS, 128), jnp.int32),
        pltpu.VMEM((E_BLOCKS, 128), jnp.int32),
        pltpu.VMEM((128, 2 * H), jnp.float32),
        pltpu.VMEM_SHARED((N_PAD, 2 * H), jnp.float32),
        pltpu.SemaphoreType.DMA,
    ],
)


def _flow_gather_body(ab_hbm, fsrc_hbm, ftgt_hbm, gs_hbm, gt_hbm,
                        sidx, tidx, rows_a, rows_b, sem_a, sem_b):
    c = lax.axis_index("c")
    s = lax.axis_index("s")
    wid = s * NC + c
    base = wid * P_BLOCKS * 128
    pltpu.sync_copy(fsrc_hbm.at[wid], sidx)
    pltpu.sync_copy(ftgt_hbm.at[wid], tidx)

    def body(j, carry):
        da = pltpu.async_copy(ab_hbm.at[sidx.at[j]], rows_a, sem_a)
        db = pltpu.async_copy(ab_hbm.at[tidx.at[j]], rows_b, sem_b)
        da.wait()
        pltpu.sync_copy(rows_a, gs_hbm.at[pl.ds(base + j * 128, 128)])
        db.wait()
        pltpu.sync_copy(rows_b, gt_hbm.at[pl.ds(base + j * 128, 128)])
        return carry

    lax.fori_loop(0, P_BLOCKS, body, 0)


_flow_gather_kernel = pl.kernel(
    _flow_gather_body,
    out_type=(jax.ShapeDtypeStruct((P_PAD, 2 * H), jnp.float32),
              jax.ShapeDtypeStruct((P_PAD, 2 * H), jnp.float32)),
    mesh=_mesh(),
    scratch_types=[
        pltpu.VMEM((P_BLOCKS, 128), jnp.int32),
        pltpu.VMEM((P_BLOCKS, 128), jnp.int32),
        pltpu.VMEM((128, 2 * H), jnp.float32),
        pltpu.VMEM((128, 2 * H), jnp.float32),
        pltpu.SemaphoreType.DMA,
        pltpu.SemaphoreType.DMA,
    ],
)


TC_BLK = 512
TC_GRID = N_PAD // TC_BLK


def _enc_body(x_ref, degp_ref, we_ref, be_ref, w1_ref, q1_ref, dinv_ref):
    cnt = degp_ref[0][:, 0:1] + degp_ref[1][:, 0:1]
    dinv = lax.rsqrt(cnt + 1.0)
    h0 = jax.nn.relu(
        jnp.dot(x_ref[...], we_ref[...], preferred_element_type=jnp.float32)
        + be_ref[...])
    q1 = jnp.dot(h0, w1_ref[...], preferred_element_type=jnp.float32) * dinv
    q1_ref[...] = jnp.concatenate(
        [q1, jnp.zeros((TC_BLK, H), jnp.float32)], axis=1)
    dinv_ref[...] = jnp.broadcast_to(dinv, (TC_BLK, 8))


def _tc_encoder(x_pad, degp, W_enc, b_enc, W1):
    return pl.pallas_call(
        _enc_body,
        grid=(TC_GRID,),
        in_specs=[
            pl.BlockSpec((TC_BLK, F_IN), lambda i: (i, 0)),
            pl.BlockSpec((NC, TC_BLK, 2 * H), lambda i: (0, i, 0)),
            pl.BlockSpec((F_IN, H), lambda i: (0, 0)),
            pl.BlockSpec((1, H), lambda i: (0, 0)),
            pl.BlockSpec((H, H), lambda i: (0, 0)),
        ],
        out_specs=[
            pl.BlockSpec((TC_BLK, 2 * H), lambda i: (i, 0)),
            pl.BlockSpec((TC_BLK, 8), lambda i: (i, 0)),
        ],
        out_shape=[
            jax.ShapeDtypeStruct((N_PAD, 2 * H), jnp.float32),
            jax.ShapeDtypeStruct((N_PAD, 8), jnp.float32),
        ],
    )(x_pad, degp, W_enc, b_enc.reshape(1, H), W1)


def _layer_body(sp_ref, q_ref, dinv_ref, b_ref, wn_ref, qn_ref):
    dinv = dinv_ref[:, 0:1]
    h = jax.nn.relu(
        (sp_ref[0][:, :H] + sp_ref[1][:, :H] + q_ref[:, :H]) * dinv
        + b_ref[...])
    qn = jnp.dot(h, wn_ref[...], preferred_element_type=jnp.float32) * dinv
    qn_ref[...] = jnp.concatenate(
        [qn, jnp.zeros((TC_BLK, H), jnp.float32)], axis=1)


def _tc_layer(sp, q, dinv, b, W_next):
    return pl.pallas_call(
        _layer_body,
        grid=(TC_GRID,),
        in_specs=[
            pl.BlockSpec((NC, TC_BLK, 2 * H), lambda i: (0, i, 0)),
            pl.BlockSpec((TC_BLK, 2 * H), lambda i: (i, 0)),
            pl.BlockSpec((TC_BLK, 8), lambda i: (i, 0)),
            pl.BlockSpec((1, H), lambda i: (0, 0)),
            pl.BlockSpec((H, H), lambda i: (0, 0)),
        ],
        out_specs=pl.BlockSpec((TC_BLK, 2 * H), lambda i: (i, 0)),
        out_shape=jax.ShapeDtypeStruct((N_PAD, 2 * H), jnp.float32),
    )(sp, q, dinv, b.reshape(1, H), W_next)


def _proj_body(sp_ref, q_ref, dinv_ref, b_ref, wc_ref, ab_ref):
    dinv = dinv_ref[:, 0:1]
    h = jax.nn.relu(
        (sp_ref[0][:, :H] + sp_ref[1][:, :H] + q_ref[:, :H]) * dinv
        + b_ref[...])
    ab_ref[...] = jnp.dot(h, wc_ref[...], preferred_element_type=jnp.float32)


def _tc_proj(sp, q, dinv, b, Wf1):
    return pl.pallas_call(
        _proj_body,
        grid=(TC_GRID,),
        in_specs=[
            pl.BlockSpec((NC, TC_BLK, 2 * H), lambda i: (0, i, 0)),
            pl.BlockSpec((TC_BLK, 2 * H), lambda i: (i, 0)),
            pl.BlockSpec((TC_BLK, 8), lambda i: (i, 0)),
            pl.BlockSpec((1, H), lambda i: (0, 0)),
            pl.BlockSpec((H, 2 * H), lambda i: (0, 0)),
        ],
        out_specs=pl.BlockSpec((TC_BLK, 2 * H), lambda i: (i, 0)),
        out_shape=jax.ShapeDtypeStruct((N_PAD, 2 * H), jnp.float32),
    )(sp, q, dinv, b.reshape(1, H),
      jnp.concatenate([Wf1[:H], Wf1[H:]], axis=1))


MLP_BLK = 1000
MLP_GRID = P // MLP_BLK


def _mlp_body(gs_ref, gt_ref, b1_ref, w2_ref, b2_ref, w3_ref, b3_ref, out_ref):
    z = jax.nn.relu(gs_ref[:, :H] + gt_ref[:, H:] + b1_ref[...])
    z2 = jax.nn.relu(
        jnp.dot(z, w2_ref[...], preferred_element_type=jnp.float32)
        + b2_ref[...])
    out_ref[...] = jnp.dot(z2, w3_ref[...],
                           preferred_element_type=jnp.float32) + b3_ref[...]


def _tc_mlp(gs, gt, bf1, Wf2, bf2, Wf3, bf3):
    return pl.pallas_call(
        _mlp_body,
        grid=(MLP_GRID,),
        in_specs=[
            pl.BlockSpec((MLP_BLK, 2 * H), lambda i: (i, 0)),
            pl.BlockSpec((MLP_BLK, 2 * H), lambda i: (i, 0)),
            pl.BlockSpec((1, H), lambda i: (0, 0)),
            pl.BlockSpec((H, H // 2), lambda i: (0, 0)),
            pl.BlockSpec((1, H // 2), lambda i: (0, 0)),
            pl.BlockSpec((H // 2, 1), lambda i: (0, 0)),
            pl.BlockSpec((1, 1), lambda i: (0, 0)),
        ],
        out_specs=pl.BlockSpec((MLP_BLK, 1), lambda i: (i, 0)),
        out_shape=jax.ShapeDtypeStruct((P, 1), jnp.float32),
    )(gs, gt, bf1.reshape(1, H), Wf2, bf2.reshape(1, H // 2), Wf3,
      bf3.reshape(1, 1))


def kernel(x, edge_index, flow_edges, W_enc, b_enc, W1, b1, W2, b2, W3, b3,
           Wf1, bf1, Wf2, bf2, Wf3, bf3):
    x_pad = jnp.concatenate(
        [x, jnp.zeros((N_PAD - N, F_IN), jnp.float32)], axis=0)

    def pad_idx(idx, length, blocks, fill):
        idx = idx.astype(jnp.int32)
        idx = jnp.concatenate(
            [idx, jnp.full((NW * blocks * 128 - length,), fill, jnp.int32)])
        return idx.reshape(NW, blocks, 128)

    src3 = pad_idx(edge_index[0], E, E_BLOCKS, N)
    dst3 = pad_idx(edge_index[1], E, E_BLOCKS, N)
    fsrc3 = pad_idx(flow_edges[0], P, P_BLOCKS, 0)
    ftgt3 = pad_idx(flow_edges[1], P, P_BLOCKS, 0)

    zeros128 = jnp.zeros((N_PAD, 2 * H), jnp.float32)
    ones128 = jnp.ones((128, 2 * H), jnp.float32)


    degp = _deg_kernel(dst3, ones128, zeros128).reshape(NC, N_PAD, 2 * H)

    q1, dinv = _tc_encoder(x_pad, degp, W_enc, b_enc, W1)

    s1 = _edge_scatter_kernel(q1, src3, dst3,
                              zeros128).reshape(NC, N_PAD, 2 * H)
    q2 = _tc_layer(s1, q1, dinv, b1, W2)
    s2 = _edge_scatter_kernel(q2, src3, dst3,
                              zeros128).reshape(NC, N_PAD, 2 * H)
    q3 = _tc_layer(s2, q2, dinv, b2, W3)
    s3 = _edge_scatter_kernel(q3, src3, dst3,
                              zeros128).reshape(NC, N_PAD, 2 * H)

    AB = _tc_proj(s3, q3, dinv, b3, Wf1)
    gs, gt = _flow_gather_kernel(AB, fsrc3, ftgt3)
    flows = _tc_mlp(gs[:P], gt[:P], bf1, Wf2, bf2, Wf3, bf3)
    return flows
